# baseline (speedup 1.0000x reference)
import dataclasses

import jax
import jax.numpy as jnp
from jax import lax
from jax.experimental import pallas as pl
from jax.experimental.pallas import tpu as pltpu
from jax.experimental.pallas import tpu_sc as plsc

NC = 2
NS = 16
NW = NC * NS
LANES = 16
N = 10000
NP = 10240
F = 128
FPT = F // NW


def _sc_mesh():
    return plsc.VectorSubcoreMesh(core_axis_name="c", subcore_axis_name="s")


def _sc_compiler_params():
    cp = pltpu.CompilerParams()
    fields = pltpu.CompilerParams.__dataclass_fields__
    if "needs_layout_passes" in fields:
        cp = dataclasses.replace(cp, needs_layout_passes=False)
    if "use_tc_tiling_on_sc" in fields:
        cp = dataclasses.replace(cp, use_tc_tiling_on_sc=False)
    return cp


@jax.jit
def _sc_deg(sd, ew):
    E = sd.shape[0]
    per_tile = E // NW
    rmain = 624
    tail = N - rmain * NS
    rbig = rmain + tail

    @pl.kernel(
        out_type=jax.ShapeDtypeStruct((NC, N, LANES), jnp.float32),
        mesh=_sc_mesh(),
        scratch_types=[
            pltpu.VMEM((per_tile,), jnp.int32),
            pltpu.VMEM((per_tile,), jnp.float32),
            pltpu.VMEM((N,), jnp.float32),
            pltpu.VMEM((LANES, rbig), jnp.float32),
            pltpu.VMEM((rbig, LANES), jnp.float32),
            pltpu.VMEM_SHARED((LANES, N), jnp.float32),
        ],
        compiler_params=_sc_compiler_params(),
    )
    def deg_kernel(sd_hbm, ew_hbm, out_hbm, sd_v, ew_v, part, slab, rows, stage):
        c = lax.axis_index("c")
        s = lax.axis_index("s")
        wid = c * NS + s
        zero = jnp.zeros((LANES,), jnp.float32)
        iota = lax.broadcasted_iota(jnp.int32, (LANES,), 0)
        c16 = jnp.full((LANES,), 16, jnp.int32)

        @pl.loop(0, N, step=LANES)
        def _(i):
            part[pl.ds(i, LANES)] = zero

        @pl.loop(0, rbig)
        def _(r):
            rows[r, pl.ds(0, LANES)] = zero

        pltpu.sync_copy(sd_hbm.at[pl.ds(wid * per_tile, per_tile)], sd_v)
        pltpu.sync_copy(ew_hbm.at[pl.ds(wid * per_tile, per_tile)], ew_v)

        @plsc.parallel_loop(0, per_tile, step=LANES, unroll=4)
        def _(k0):
            d16 = lax.shift_right_logical(sd_v[pl.ds(k0, LANES)], c16)
            w16 = ew_v[pl.ds(k0, LANES)]
            plsc.addupdate_scatter(part, [d16], w16)

        pltpu.sync_copy(part, stage.at[s])
        plsc.subcore_barrier()

        base = s * rmain
        pltpu.sync_copy(stage.at[pl.ds(0, LANES), pl.ds(base, rmain)],
                        slab.at[pl.ds(0, LANES), pl.ds(0, rmain)])

        @pl.when(s == NS - 1)
        def _():
            pltpu.sync_copy(stage.at[pl.ds(0, LANES), pl.ds(NS * rmain, tail)],
                            slab.at[pl.ds(0, LANES), pl.ds(rmain, tail)])

        def reduce_chunk(i):
            tot = slab[0, pl.ds(i, LANES)]
            for k in range(1, LANES):
                tot = tot + slab[k, pl.ds(i, LANES)]
            plsc.store_scatter(rows, [i + iota, jnp.zeros((LANES,), jnp.int32)],
                               tot)

        @pl.loop(0, rmain, step=LANES)
        def _(i):
            reduce_chunk(i)

        @pl.when(s == NS - 1)
        def _():
            reduce_chunk(rmain)

        pltpu.sync_copy(rows.at[pl.ds(0, rmain)],
                        out_hbm.at[c, pl.ds(base, rmain)])

        @pl.when(s == NS - 1)
        def _():
            pltpu.sync_copy(rows.at[pl.ds(rmain, tail)],
                            out_hbm.at[c, pl.ds(NS * rmain, tail)])

    return deg_kernel(sd, ew)


@jax.jit
def _sc_edge_agg(yt, sd, ew):
    E = sd.shape[0]
    K = 8000
    NCH = E // K

    @pl.kernel(
        out_type=jax.ShapeDtypeStruct((F, NP), jnp.float32),
        mesh=_sc_mesh(),
        scratch_types=[
            pltpu.VMEM((FPT // 2, NP), jnp.int32),
            pltpu.VMEM((FPT, NP), jnp.float32),
            pltpu.VMEM((2, K), jnp.int32),
            pltpu.VMEM((2, K), jnp.float32),
            pltpu.SemaphoreType.DMA,
            pltpu.SemaphoreType.DMA,
        ],
        compiler_params=_sc_compiler_params(),
    )
    def agg_kernel(yt_hbm, sd_hbm, ew_hbm, out_hbm,
                   slab, acc, sd_v, ew_v, sem0, sem1):
        c = lax.axis_index("c")
        s = lax.axis_index("s")
        g0 = (c * NS + s) * (FPT // 2)
        zero = jnp.zeros((LANES,), jnp.float32)
        sems = (sem0, sem1)
        c16 = jnp.full((LANES,), 16, jnp.int32)
        chi = jnp.full((LANES,), -65536, jnp.int32)
        clo = jnp.full((LANES,), 65535, jnp.int32)

        def chunk_copies(b, ci):
            e0 = ci * K
            return (
                pltpu.make_async_copy(sd_hbm.at[pl.ds(e0, K)],
                                      sd_v.at[b], sems[b]),
                pltpu.make_async_copy(ew_hbm.at[pl.ds(e0, K)],
                                      ew_v.at[b], sems[b]),
            )

        def start_set(b, ci):
            for cp in chunk_copies(b, ci):
                cp.start()

        def wait_set(b):
            for cp in chunk_copies(b, 0):
                cp.wait()

        def compute_set(b):
            @plsc.parallel_loop(0, K, step=LANES, unroll=8)
            def _(k0):
                sd16 = sd_v[b, pl.ds(k0, LANES)]
                s16 = jnp.bitwise_and(sd16, clo)
                d16 = lax.shift_right_logical(sd16, c16)
                w16 = ew_v[b, pl.ds(k0, LANES)]
                for p in range(FPT // 2):
                    wv = plsc.load_gather(slab.at[p], [s16])
                    vlo = plsc.bitcast(jnp.left_shift(wv, c16), jnp.float32)
                    vhi = plsc.bitcast(jnp.bitwise_and(wv, chi), jnp.float32)
                    plsc.addupdate_scatter(acc.at[p], [d16], vlo * w16)
                    plsc.addupdate_scatter(acc.at[FPT // 2 + p], [d16],
                                           vhi * w16)

        pltpu.sync_copy(yt_hbm.at[pl.ds(g0, FPT // 2)], slab)

        for f in range(FPT):
            @plsc.parallel_loop(0, NP, step=LANES)
            def _(i):
                acc[f, pl.ds(i, LANES)] = zero

        start_set(0, 0)

        @pl.loop(0, NCH, step=2)
        def _(ci):
            start_set(1, ci + 1)
            wait_set(0)
            compute_set(0)
            start_set(0, jnp.minimum(ci + 2, NCH - 2))
            wait_set(1)
            compute_set(1)

        wait_set(0)
        pltpu.sync_copy(acc.at[pl.ds(0, FPT // 2)],
                        out_hbm.at[pl.ds(g0, FPT // 2)])
        pltpu.sync_copy(acc.at[pl.ds(FPT // 2, FPT // 2)],
                        out_hbm.at[pl.ds(F // 2 + g0, FPT // 2)])

    return agg_kernel(yt, sd, ew)


_BR = 1024
_NB = (N + _BR - 1) // _BR


def _deg_to_d(dp_block):
    deg = 1.0 + jnp.sum(dp_block, axis=(0, 2))
    return lax.rsqrt(deg)


def _pack_t(y):
    bits = jax.lax.bitcast_convert_type(y.astype(jnp.bfloat16),
                                        jnp.uint16).astype(jnp.uint32)
    w = bits[:, : F // 2] | (bits[:, F // 2:] << 16)
    return jax.lax.bitcast_convert_type(w, jnp.int32).T


def _pre_body(x_ref, w_ref, dp_ref, y_ref, yt_ref):
    d = _deg_to_d(dp_ref[...])
    xw = jnp.dot(x_ref[...], w_ref[...], preferred_element_type=jnp.float32)
    y = xw * d[:, None]
    y_ref[...] = y
    yt_ref[...] = _pack_t(y)


def _mid_body(acc_ref, y_ref, dp_ref, b_ref, w_ref, y2_ref, y2t_ref):
    d = _deg_to_d(dp_ref[...])
    h = (acc_ref[...].T + y_ref[...]) * d[:, None] + b_ref[...][None, :]
    h = jnp.maximum(h, 0.0)
    y2 = jnp.dot(h, w_ref[...], preferred_element_type=jnp.float32) * d[:, None]
    y2_ref[...] = y2
    y2t_ref[...] = _pack_t(y2)


def _post_body(acc_ref, y_ref, dp_ref, b_ref, out_ref):
    d = _deg_to_d(dp_ref[...])
    h = (acc_ref[...].T + y_ref[...]) * d[:, None] + b_ref[...][None, :]
    out_ref[...] = jnp.maximum(h, 0.0)


_spec_rows = pl.BlockSpec((_BR, F), lambda i: (i, 0))
_spec_rows_t = pl.BlockSpec((F, _BR), lambda i: (0, i))
_spec_packed = pl.BlockSpec((F // 2, _BR), lambda i: (0, i))
_spec_dp = pl.BlockSpec((NC, _BR, LANES), lambda i: (0, i, 0))
_spec_w = pl.BlockSpec((F, F), lambda i: (0, 0))
_spec_b = pl.BlockSpec((F,), lambda i: (0,))


@jax.jit
def _tc_pre(x, W, deg_par):
    return pl.pallas_call(
        _pre_body,
        grid=(_NB,),
        in_specs=[_spec_rows, _spec_w, _spec_dp],
        out_specs=[_spec_rows, _spec_packed],
        out_shape=[jax.ShapeDtypeStruct((N, F), jnp.float32),
                   jax.ShapeDtypeStruct((F // 2, NP), jnp.int32)],
    )(x, W, deg_par)


@jax.jit
def _tc_mid(acct, y, deg_par, b, W):
    return pl.pallas_call(
        _mid_body,
        grid=(_NB,),
        in_specs=[_spec_rows_t, _spec_rows, _spec_dp, _spec_b, _spec_w],
        out_specs=[_spec_rows, _spec_packed],
        out_shape=[jax.ShapeDtypeStruct((N, F), jnp.float32),
                   jax.ShapeDtypeStruct((F // 2, NP), jnp.int32)],
    )(acct, y, deg_par, b, W)


@jax.jit
def _tc_post(acct, y, deg_par, b):
    return pl.pallas_call(
        _post_body,
        grid=(_NB,),
        in_specs=[_spec_rows_t, _spec_rows, _spec_dp, _spec_b],
        out_specs=_spec_rows,
        out_shape=jax.ShapeDtypeStruct((N, F), jnp.float32),
    )(acct, y, deg_par, b)


def kernel(x, edge_index, edge_weight, W1, b1, W2, b2):
    src = edge_index[0].astype(jnp.int32)
    dst = edge_index[1].astype(jnp.int32)
    sd = src | (dst << 16)
    ew = edge_weight.astype(jnp.float32)

    deg_par = _sc_deg(sd, ew)
    y1, y1t = _tc_pre(x, W1, deg_par)
    acc1t = _sc_edge_agg(y1t, sd, ew)
    y2, y2t = _tc_mid(acc1t, y1, deg_par, b1, W2)
    acc2t = _sc_edge_agg(y2t, sd, ew)
    return _tc_post(acc2t, y2, deg_par, b2)


# speedup vs baseline: 1.0171x; 1.0171x over previous
import dataclasses

import jax
import jax.numpy as jnp
from jax import lax
from jax.experimental import pallas as pl
from jax.experimental.pallas import tpu as pltpu
from jax.experimental.pallas import tpu_sc as plsc

NC = 2
NS = 16
NW = NC * NS
LANES = 16
N = 10000
NP = 10240
F = 128
FPT = F // NW


def _sc_mesh():
    return plsc.VectorSubcoreMesh(core_axis_name="c", subcore_axis_name="s")


def _sc_compiler_params():
    cp = pltpu.CompilerParams()
    fields = pltpu.CompilerParams.__dataclass_fields__
    if "needs_layout_passes" in fields:
        cp = dataclasses.replace(cp, needs_layout_passes=False)
    if "use_tc_tiling_on_sc" in fields:
        cp = dataclasses.replace(cp, use_tc_tiling_on_sc=False)
    return cp


@jax.jit
def _sc_deg(sd, ew):
    E = sd.shape[0]
    per_tile = E // NW
    rmain = 624
    tail = N - rmain * NS
    rbig = rmain + tail

    @pl.kernel(
        out_type=jax.ShapeDtypeStruct((NC, N, LANES), jnp.float32),
        mesh=_sc_mesh(),
        scratch_types=[
            pltpu.VMEM((per_tile,), jnp.int32),
            pltpu.VMEM((per_tile,), jnp.float32),
            pltpu.VMEM((N,), jnp.float32),
            pltpu.VMEM((LANES, rbig), jnp.float32),
            pltpu.VMEM((rbig, LANES), jnp.float32),
            pltpu.VMEM_SHARED((LANES, N), jnp.float32),
        ],
        compiler_params=_sc_compiler_params(),
    )
    def deg_kernel(sd_hbm, ew_hbm, out_hbm, sd_v, ew_v, part, slab, rows, stage):
        c = lax.axis_index("c")
        s = lax.axis_index("s")
        wid = c * NS + s
        zero = jnp.zeros((LANES,), jnp.float32)
        iota = lax.broadcasted_iota(jnp.int32, (LANES,), 0)
        c16 = jnp.full((LANES,), 16, jnp.int32)

        @pl.loop(0, N, step=LANES)
        def _(i):
            part[pl.ds(i, LANES)] = zero

        @pl.loop(0, rbig)
        def _(r):
            rows[r, pl.ds(0, LANES)] = zero

        pltpu.sync_copy(sd_hbm.at[pl.ds(wid * per_tile, per_tile)], sd_v)
        pltpu.sync_copy(ew_hbm.at[pl.ds(wid * per_tile, per_tile)], ew_v)

        @plsc.parallel_loop(0, per_tile, step=LANES, unroll=4)
        def _(k0):
            d16 = lax.shift_right_logical(sd_v[pl.ds(k0, LANES)], c16)
            w16 = ew_v[pl.ds(k0, LANES)]
            plsc.addupdate_scatter(part, [d16], w16)

        pltpu.sync_copy(part, stage.at[s])
        plsc.subcore_barrier()

        base = s * rmain
        pltpu.sync_copy(stage.at[pl.ds(0, LANES), pl.ds(base, rmain)],
                        slab.at[pl.ds(0, LANES), pl.ds(0, rmain)])

        @pl.when(s == NS - 1)
        def _():
            pltpu.sync_copy(stage.at[pl.ds(0, LANES), pl.ds(NS * rmain, tail)],
                            slab.at[pl.ds(0, LANES), pl.ds(rmain, tail)])

        def reduce_chunk(i):
            tot = slab[0, pl.ds(i, LANES)]
            for k in range(1, LANES):
                tot = tot + slab[k, pl.ds(i, LANES)]
            plsc.store_scatter(rows, [i + iota, jnp.zeros((LANES,), jnp.int32)],
                               tot)

        @pl.loop(0, rmain, step=LANES)
        def _(i):
            reduce_chunk(i)

        @pl.when(s == NS - 1)
        def _():
            reduce_chunk(rmain)

        pltpu.sync_copy(rows.at[pl.ds(0, rmain)],
                        out_hbm.at[c, pl.ds(base, rmain)])

        @pl.when(s == NS - 1)
        def _():
            pltpu.sync_copy(rows.at[pl.ds(rmain, tail)],
                            out_hbm.at[c, pl.ds(NS * rmain, tail)])

    return deg_kernel(sd, ew)


@jax.jit
def _sc_edge_agg(yt, sd, ew):
    E = sd.shape[0]
    K = 8000
    NCH = E // K

    @pl.kernel(
        out_type=jax.ShapeDtypeStruct((F, NP), jnp.float32),
        mesh=_sc_mesh(),
        scratch_types=[
            pltpu.VMEM((FPT // 2, NP), jnp.int32),
            pltpu.VMEM((FPT, NP), jnp.float32),
            pltpu.VMEM((2, K), jnp.int32),
            pltpu.VMEM((2, K), jnp.float32),
            pltpu.SemaphoreType.DMA,
            pltpu.SemaphoreType.DMA,
        ],
        compiler_params=_sc_compiler_params(),
    )
    def agg_kernel(yt_hbm, sd_hbm, ew_hbm, out_hbm,
                   slab, acc, sd_v, ew_v, sem0, sem1):
        c = lax.axis_index("c")
        s = lax.axis_index("s")
        g0 = (c * NS + s) * (FPT // 2)
        zero = jnp.zeros((LANES,), jnp.float32)
        sems = (sem0, sem1)
        c16 = jnp.full((LANES,), 16, jnp.int32)
        chi = jnp.full((LANES,), -65536, jnp.int32)
        clo = jnp.full((LANES,), 65535, jnp.int32)

        def chunk_copies(b, ci):
            e0 = ci * K
            return (
                pltpu.make_async_copy(sd_hbm.at[pl.ds(e0, K)],
                                      sd_v.at[b], sems[b]),
                pltpu.make_async_copy(ew_hbm.at[pl.ds(e0, K)],
                                      ew_v.at[b], sems[b]),
            )

        def start_set(b, ci):
            for cp in chunk_copies(b, ci):
                cp.start()

        def wait_set(b):
            for cp in chunk_copies(b, 0):
                cp.wait()

        def compute_set(b):
            @plsc.parallel_loop(0, K, step=LANES, unroll=4)
            def _(k0):
                sd16 = sd_v[b, pl.ds(k0, LANES)]
                s16 = jnp.bitwise_and(sd16, clo)
                d16 = lax.shift_right_logical(sd16, c16)
                w16 = ew_v[b, pl.ds(k0, LANES)]
                for p in range(FPT // 2):
                    wv = plsc.load_gather(slab.at[p], [s16])
                    vlo = plsc.bitcast(jnp.left_shift(wv, c16), jnp.float32)
                    vhi = plsc.bitcast(jnp.bitwise_and(wv, chi), jnp.float32)
                    plsc.addupdate_scatter(acc.at[p], [d16], vlo * w16)
                    plsc.addupdate_scatter(acc.at[FPT // 2 + p], [d16],
                                           vhi * w16)

        pltpu.sync_copy(yt_hbm.at[pl.ds(g0, FPT // 2)], slab)

        for f in range(FPT):
            @plsc.parallel_loop(0, NP, step=LANES)
            def _(i):
                acc[f, pl.ds(i, LANES)] = zero

        start_set(0, 0)

        @pl.loop(0, NCH, step=2)
        def _(ci):
            start_set(1, ci + 1)
            wait_set(0)
            compute_set(0)
            start_set(0, jnp.minimum(ci + 2, NCH - 2))
            wait_set(1)
            compute_set(1)

        wait_set(0)
        pltpu.sync_copy(acc.at[pl.ds(0, FPT // 2)],
                        out_hbm.at[pl.ds(g0, FPT // 2)])
        pltpu.sync_copy(acc.at[pl.ds(FPT // 2, FPT // 2)],
                        out_hbm.at[pl.ds(F // 2 + g0, FPT // 2)])

    return agg_kernel(yt, sd, ew)


_BR = 1024
_NB = (N + _BR - 1) // _BR


def _deg_to_d(dp_block):
    deg = 1.0 + jnp.sum(dp_block, axis=(0, 2))
    return lax.rsqrt(deg)


def _pack_t(y):
    bits = jax.lax.bitcast_convert_type(y.astype(jnp.bfloat16),
                                        jnp.uint16).astype(jnp.uint32)
    w = bits[:, : F // 2] | (bits[:, F // 2:] << 16)
    return jax.lax.bitcast_convert_type(w, jnp.int32).T


def _pre_body(x_ref, w_ref, dp_ref, y_ref, yt_ref):
    d = _deg_to_d(dp_ref[...])
    xw = jnp.dot(x_ref[...], w_ref[...], preferred_element_type=jnp.float32)
    y = xw * d[:, None]
    y_ref[...] = y
    yt_ref[...] = _pack_t(y)


def _mid_body(acc_ref, y_ref, dp_ref, b_ref, w_ref, y2_ref, y2t_ref):
    d = _deg_to_d(dp_ref[...])
    h = (acc_ref[...].T + y_ref[...]) * d[:, None] + b_ref[...][None, :]
    h = jnp.maximum(h, 0.0)
    y2 = jnp.dot(h, w_ref[...], preferred_element_type=jnp.float32) * d[:, None]
    y2_ref[...] = y2
    y2t_ref[...] = _pack_t(y2)


def _post_body(acc_ref, y_ref, dp_ref, b_ref, out_ref):
    d = _deg_to_d(dp_ref[...])
    h = (acc_ref[...].T + y_ref[...]) * d[:, None] + b_ref[...][None, :]
    out_ref[...] = jnp.maximum(h, 0.0)


_spec_rows = pl.BlockSpec((_BR, F), lambda i: (i, 0))
_spec_rows_t = pl.BlockSpec((F, _BR), lambda i: (0, i))
_spec_packed = pl.BlockSpec((F // 2, _BR), lambda i: (0, i))
_spec_dp = pl.BlockSpec((NC, _BR, LANES), lambda i: (0, i, 0))
_spec_w = pl.BlockSpec((F, F), lambda i: (0, 0))
_spec_b = pl.BlockSpec((F,), lambda i: (0,))


@jax.jit
def _tc_pre(x, W, deg_par):
    return pl.pallas_call(
        _pre_body,
        grid=(_NB,),
        in_specs=[_spec_rows, _spec_w, _spec_dp],
        out_specs=[_spec_rows, _spec_packed],
        out_shape=[jax.ShapeDtypeStruct((N, F), jnp.float32),
                   jax.ShapeDtypeStruct((F // 2, NP), jnp.int32)],
    )(x, W, deg_par)


@jax.jit
def _tc_mid(acct, y, deg_par, b, W):
    return pl.pallas_call(
        _mid_body,
        grid=(_NB,),
        in_specs=[_spec_rows_t, _spec_rows, _spec_dp, _spec_b, _spec_w],
        out_specs=[_spec_rows, _spec_packed],
        out_shape=[jax.ShapeDtypeStruct((N, F), jnp.float32),
                   jax.ShapeDtypeStruct((F // 2, NP), jnp.int32)],
    )(acct, y, deg_par, b, W)


@jax.jit
def _tc_post(acct, y, deg_par, b):
    return pl.pallas_call(
        _post_body,
        grid=(_NB,),
        in_specs=[_spec_rows_t, _spec_rows, _spec_dp, _spec_b],
        out_specs=_spec_rows,
        out_shape=jax.ShapeDtypeStruct((N, F), jnp.float32),
    )(acct, y, deg_par, b)


def kernel(x, edge_index, edge_weight, W1, b1, W2, b2):
    src = edge_index[0].astype(jnp.int32)
    dst = edge_index[1].astype(jnp.int32)
    sd = src | (dst << 16)
    ew = edge_weight.astype(jnp.float32)

    deg_par = _sc_deg(sd, ew)
    y1, y1t = _tc_pre(x, W1, deg_par)
    acc1t = _sc_edge_agg(y1t, sd, ew)
    y2, y2t = _tc_mid(acc1t, y1, deg_par, b1, W2)
    acc2t = _sc_edge_agg(y2t, sd, ew)
    return _tc_post(acc2t, y2, deg_par, b2)
